# baseline (speedup 1.0000x reference)
import functools

import jax
import jax.numpy as jnp
from jax import lax
from jax.experimental import pallas as pl
from jax.experimental.pallas import tpu as pltpu
from jax.experimental.pallas import tpu_sc as plsc

N_NODES = 100000
D_IN = 128
NUM_GRAPHS = 512

F0 = 29504

NW = 32
ROWS_W = 2240
CHUNK = 448
NCHUNK = ROWS_W // CHUNK
GROUPS = CHUNK // 16
CW = CHUNK * D_IN
LAST_START = N_NODES - CHUNK

BN = 3688
NSTEPS = F0 // BN
WIN = 128


def _sc_kernel(xf, lp, bid, w, out, bufx0, bufx1, buflp0, buflp1, bufid0,
               bufid1, wbuf, acc, ybuf,
               sx0, sx1, sl0, sl1, si0, si1):
    core = lax.axis_index("c")
    sid = lax.axis_index("s")
    wid = core * 16 + sid

    pltpu.sync_copy(w, wbuf)
    iota = lax.broadcasted_iota(jnp.int32, (16,), 0)
    col = iota * 16
    wv = [wbuf[pl.ds(16 * k, 16)] for k in range(8)]

    zero16 = jnp.zeros((16,), jnp.float32)
    for r in range(32):
        acc[r, :] = zero16

    bufx = (bufx0, bufx1)
    buflp = (buflp0, buflp1)
    bufid = (bufid0, bufid1)
    semx = (sx0, sx1)
    seml = (sl0, sl1)
    semi = (si0, si1)

    def chunk_start(c):
        nominal = F0 + wid * ROWS_W + c * CHUNK
        start = jnp.minimum(nominal, LAST_START)
        g_lo = (nominal - start) // 16
        return start, g_lo

    def launch(c):
        slot = c % 2
        start, _ = chunk_start(c)
        dx = pltpu.async_copy(xf.at[pl.ds(start * D_IN, CW)], bufx[slot],
                              semx[slot])
        dl = pltpu.async_copy(lp.at[pl.ds(start, CHUNK)], buflp[slot],
                              seml[slot])
        di = pltpu.async_copy(bid.at[pl.ds(start, CHUNK)], bufid[slot],
                              semi[slot])
        return dx, dl, di

    pending = launch(0)
    for c in range(NCHUNK):
        slot = c % 2
        for d in pending:
            d.wait()
        if c + 1 < NCHUNK:
            pending = launch(c + 1)
        _, g_lo = chunk_start(c)
        bx, bl, bi = bufx[slot], buflp[slot], bufid[slot]

        def row_body(r4, _):
            base = r4 * (4 * D_IN)
            for r in range(4):
                m = [bx[pl.ds(base + r * D_IN + 16 * k, 16)] * wv[k]
                     for k in range(8)]
                a = ((m[0] + m[1]) + (m[2] + m[3])) + \
                    ((m[4] + m[5]) + (m[6] + m[7]))
                ybuf[pl.ds(r4 * 64 + r * 16, 16)] = a
            return 0

        lax.fori_loop(0, CHUNK // 4, row_body, 0)

        def group_body(g, _):
            rb = g * 16
            yb = rb * 16
            g4 = []
            for q in range(4):
                t = [plsc.load_gather(ybuf, [yb + col + 4 * q + j])
                     for j in range(4)]
                g4.append((t[0] + t[1]) + (t[2] + t[3]))
            s = (g4[0] + g4[1]) + (g4[2] + g4[3])
            lpv = bl[pl.ds(rb, 16)]
            s = s * jnp.exp(lpv)
            ids = bi[pl.ds(rb, 16)]
            cs = plsc.cumsum(s)
            ids_next = plsc.load_gather(bi, [rb + jnp.minimum(iota + 1, 15)])
            is_end = (ids != ids_next) | (iota == 15)
            plsc.addupdate_scatter(acc, [ids >> 4, ids & 15], cs, mask=is_end)
            m2 = is_end & (iota < 15)
            plsc.addupdate_scatter(acc, [ids_next >> 4, ids_next & 15], -cs,
                                   mask=m2)
            return 0

        lax.fori_loop(g_lo, GROUPS, group_body, 0)

    pltpu.sync_copy(acc, out.at[wid])


def _tc_kernel(meta_ref, x_ref, lp_ref, bid_ref, w_ref, out_ref):
    i = pl.program_id(0)
    base = meta_ref[i, 0]
    isfull = meta_ref[i, 1]

    xw = jnp.dot(x_ref[...], w_ref[...], preferred_element_type=jnp.float32)
    s = xw * jnp.exp(lp_ref[...])

    @pl.when(i == 0)
    def _init():
        out_ref[...] = jnp.zeros((NUM_GRAPHS, 1), jnp.float32)

    @pl.when(isfull == 0)
    def _windowed():
        gids = base + jax.lax.broadcasted_iota(jnp.int32, (1, WIN), 1)
        mask = (bid_ref[...] == gids).astype(jnp.float32)
        contrib = jax.lax.dot_general(
            mask, s, (((0,), (0,)), ((), ())),
            preferred_element_type=jnp.float32)
        out_ref[pl.ds(base, WIN), :] += contrib

    @pl.when(isfull != 0)
    def _full():
        gids = jax.lax.broadcasted_iota(jnp.int32, (1, NUM_GRAPHS), 1)
        mask = (bid_ref[...] == gids).astype(jnp.float32)
        contrib = jax.lax.dot_general(
            mask, s, (((0,), (0,)), ((), ())),
            preferred_element_type=jnp.float32)
        out_ref[...] += contrib


def _merge_kernel(psc_ref, ptc_ref, b_ref, out_ref):
    out_ref[...] = (jnp.sum(psc_ref[...], axis=0, keepdims=True)
                    + ptc_ref[...] + b_ref[...])


@jax.jit
def kernel(x, node_logprob, batch_ids, y, W, b):
    xf = x.reshape(N_NODES * D_IN)
    bid = batch_ids.astype(jnp.int32)
    w = W.reshape(D_IN)

    mesh = plsc.VectorSubcoreMesh(core_axis_name="c", subcore_axis_name="s")
    sc = functools.partial(
        pl.kernel,
        out_type=jax.ShapeDtypeStruct((NW, 32, 16), jnp.float32),
        mesh=mesh,
        compiler_params=pltpu.CompilerParams(needs_layout_passes=False),
        scratch_types=[
            pltpu.VMEM((CW,), jnp.float32),
            pltpu.VMEM((CW,), jnp.float32),
            pltpu.VMEM((CHUNK,), jnp.float32),
            pltpu.VMEM((CHUNK,), jnp.float32),
            pltpu.VMEM((CHUNK,), jnp.int32),
            pltpu.VMEM((CHUNK,), jnp.int32),
            pltpu.VMEM((D_IN,), jnp.float32),
            pltpu.VMEM((32, 16), jnp.float32),
            pltpu.VMEM((CHUNK * 16,), jnp.float32),
            pltpu.SemaphoreType.DMA,
            pltpu.SemaphoreType.DMA,
            pltpu.SemaphoreType.DMA,
            pltpu.SemaphoreType.DMA,
            pltpu.SemaphoreType.DMA,
            pltpu.SemaphoreType.DMA,
        ],
    )(_sc_kernel)
    partials_sc = sc(xf, node_logprob, bid, w)

    lp_tc = node_logprob[:F0].reshape(F0, 1)
    bid_tc = bid[:F0].reshape(F0, 1)
    starts = bid_tc[::BN, 0]
    ends = bid_tc[BN - 1 :: BN, 0]
    wbase = jnp.minimum((starts // 8) * 8, NUM_GRAPHS - WIN)
    isfull = (ends >= wbase + WIN).astype(jnp.int32)
    meta = jnp.stack([wbase, isfull], axis=1)

    partial_tc = pl.pallas_call(
        _tc_kernel,
        grid_spec=pltpu.PrefetchScalarGridSpec(
            num_scalar_prefetch=1,
            grid=(NSTEPS,),
            in_specs=[
                pl.BlockSpec((BN, D_IN), lambda i, m: (i, 0)),
                pl.BlockSpec((BN, 1), lambda i, m: (i, 0)),
                pl.BlockSpec((BN, 1), lambda i, m: (i, 0)),
                pl.BlockSpec((D_IN, 1), lambda i, m: (0, 0)),
            ],
            out_specs=pl.BlockSpec((NUM_GRAPHS, 1), lambda i, m: (0, 0)),
        ),
        out_shape=jax.ShapeDtypeStruct((NUM_GRAPHS, 1), jnp.float32),
    )(meta, x[:F0], lp_tc, bid_tc, W)

    pred = pl.pallas_call(
        _merge_kernel,
        out_shape=jax.ShapeDtypeStruct((1, NUM_GRAPHS), jnp.float32),
    )(partials_sc.reshape(NW, NUM_GRAPHS), partial_tc.reshape(1, NUM_GRAPHS),
      b.reshape(1, 1))
    return (pred.reshape(NUM_GRAPHS, 1), y)


# speedup vs baseline: 1.0627x; 1.0627x over previous
import functools

import jax
import jax.numpy as jnp
from jax import lax
from jax.experimental import pallas as pl
from jax.experimental.pallas import tpu as pltpu
from jax.experimental.pallas import tpu_sc as plsc

N_NODES = 100000
D_IN = 128
NUM_GRAPHS = 512

NW = 32
ROWS_W = 3136
CHUNK = 448
NCHUNK = ROWS_W // CHUNK
GROUPS = CHUNK // 16
CW = CHUNK * D_IN
LAST_START = N_NODES - CHUNK


def _sc_kernel(xf, lp, bid, w, out, bufx0, bufx1, buflp0, buflp1, bufid0,
               bufid1, wbuf, acc, ybuf,
               sx0, sx1, sl0, sl1, si0, si1):
    core = lax.axis_index("c")
    sid = lax.axis_index("s")
    wid = core * 16 + sid

    pltpu.sync_copy(w, wbuf)
    iota = lax.broadcasted_iota(jnp.int32, (16,), 0)
    col = iota * 16
    wv = [wbuf[pl.ds(16 * k, 16)] for k in range(8)]

    zero16 = jnp.zeros((16,), jnp.float32)
    for r in range(32):
        acc[r, :] = zero16

    bufx = (bufx0, bufx1)
    buflp = (buflp0, buflp1)
    bufid = (bufid0, bufid1)
    semx = (sx0, sx1)
    seml = (sl0, sl1)
    semi = (si0, si1)

    def chunk_start(c):
        nominal = wid * ROWS_W + c * CHUNK
        start = jnp.minimum(nominal, LAST_START)
        g_lo = (nominal - start) // 16
        return start, g_lo

    def launch(c):
        slot = c % 2
        start, _ = chunk_start(c)
        half = CW // 2
        dx0 = pltpu.async_copy(xf.at[pl.ds(start * D_IN, half)],
                               bufx[slot].at[pl.ds(0, half)], semx[slot])
        dx1 = pltpu.async_copy(xf.at[pl.ds(start * D_IN + half, half)],
                               bufx[slot].at[pl.ds(half, half)], seml[slot])
        dl = pltpu.async_copy(lp.at[pl.ds(start, CHUNK)], buflp[slot],
                              semi[slot])
        di = pltpu.async_copy(bid.at[pl.ds(start, CHUNK)], bufid[slot],
                              semi[slot])
        return dx0, dx1, dl, di

    pending = launch(0)
    for c in range(NCHUNK):
        slot = c % 2
        for d in pending:
            d.wait()
        if c + 1 < NCHUNK:
            pending = launch(c + 1)
        _, g_lo = chunk_start(c)
        bx, bl, bi = bufx[slot], buflp[slot], bufid[slot]

        def row_body(r4, _):
            base = r4 * (4 * D_IN)
            for r in range(4):
                m = [bx[pl.ds(base + r * D_IN + 16 * k, 16)] * wv[k]
                     for k in range(8)]
                a = ((m[0] + m[1]) + (m[2] + m[3])) + \
                    ((m[4] + m[5]) + (m[6] + m[7]))
                ybuf[pl.ds(r4 * 64 + r * 16, 16)] = a
            return 0

        lax.fori_loop(0, CHUNK // 4, row_body, 0)

        def group_body(g, _):
            rb = g * 16
            yb = rb * 16
            g4 = []
            for q in range(4):
                t = [plsc.load_gather(ybuf, [yb + col + 4 * q + j])
                     for j in range(4)]
                g4.append((t[0] + t[1]) + (t[2] + t[3]))
            s = (g4[0] + g4[1]) + (g4[2] + g4[3])
            lpv = bl[pl.ds(rb, 16)]
            s = s * jnp.exp(lpv)
            ids = bi[pl.ds(rb, 16)]
            cs = plsc.cumsum(s)
            ids_next = plsc.load_gather(bi, [rb + jnp.minimum(iota + 1, 15)])
            is_end = (ids != ids_next) | (iota == 15)
            plsc.addupdate_scatter(acc, [ids >> 4, ids & 15], cs, mask=is_end)
            m2 = is_end & (iota < 15)
            plsc.addupdate_scatter(acc, [ids_next >> 4, ids_next & 15], -cs,
                                   mask=m2)
            return 0

        lax.fori_loop(g_lo, GROUPS, group_body, 0)

    pltpu.sync_copy(acc, out.at[wid])


def _merge_kernel(p_ref, b_ref, out_ref):
    out_ref[...] = jnp.sum(p_ref[...], axis=0, keepdims=True) + b_ref[...]


@jax.jit
def kernel(x, node_logprob, batch_ids, y, W, b):
    xf = x.reshape(N_NODES * D_IN)
    bid = batch_ids.astype(jnp.int32)
    w = W.reshape(D_IN)

    mesh = plsc.VectorSubcoreMesh(core_axis_name="c", subcore_axis_name="s")
    sc = functools.partial(
        pl.kernel,
        out_type=jax.ShapeDtypeStruct((NW, 32, 16), jnp.float32),
        mesh=mesh,
        compiler_params=pltpu.CompilerParams(needs_layout_passes=False),
        scratch_types=[
            pltpu.VMEM((CW,), jnp.float32),
            pltpu.VMEM((CW,), jnp.float32),
            pltpu.VMEM((CHUNK,), jnp.float32),
            pltpu.VMEM((CHUNK,), jnp.float32),
            pltpu.VMEM((CHUNK,), jnp.int32),
            pltpu.VMEM((CHUNK,), jnp.int32),
            pltpu.VMEM((D_IN,), jnp.float32),
            pltpu.VMEM((32, 16), jnp.float32),
            pltpu.VMEM((CHUNK * 16,), jnp.float32),
            pltpu.SemaphoreType.DMA,
            pltpu.SemaphoreType.DMA,
            pltpu.SemaphoreType.DMA,
            pltpu.SemaphoreType.DMA,
            pltpu.SemaphoreType.DMA,
            pltpu.SemaphoreType.DMA,
        ],
    )(_sc_kernel)
    partials = sc(xf, node_logprob, bid, w)

    pred = pl.pallas_call(
        _merge_kernel,
        out_shape=jax.ShapeDtypeStruct((1, NUM_GRAPHS), jnp.float32),
    )(partials.reshape(NW, NUM_GRAPHS), b.reshape(1, 1))
    return (pred.reshape(NUM_GRAPHS, 1), y)


# speedup vs baseline: 1.0677x; 1.0047x over previous
import functools

import jax
import jax.numpy as jnp
from jax import lax
from jax.experimental import pallas as pl
from jax.experimental.pallas import tpu as pltpu
from jax.experimental.pallas import tpu_sc as plsc

N_NODES = 100000
D_IN = 128
NUM_GRAPHS = 512

NW = 32
ROWS_W = 3136
CHUNK = 448
NCHUNK = ROWS_W // CHUNK
GROUPS = CHUNK // 16
CW = CHUNK * D_IN
LAST_START = N_NODES - CHUNK


def _sc_kernel(xf, lp, bid, w, out, bufx0, bufx1, buflp0, buflp1, bufid0,
               bufid1, wbuf, acc, ybuf,
               sx0, sx1, sl0, sl1, si0, si1):
    core = lax.axis_index("c")
    sid = lax.axis_index("s")
    wid = core * 16 + sid

    pltpu.sync_copy(w, wbuf)
    iota = lax.broadcasted_iota(jnp.int32, (16,), 0)
    col = iota * 16
    wv = [wbuf[pl.ds(16 * k, 16)] for k in range(8)]

    zero16 = jnp.zeros((16,), jnp.float32)
    for r in range(32):
        acc[r, :] = zero16

    bufx = (bufx0, bufx1)
    buflp = (buflp0, buflp1)
    bufid = (bufid0, bufid1)
    semx = (sx0, sx1)
    seml = (sl0, sl1)
    semi = (si0, si1)

    def chunk_start(c):
        nominal = wid * ROWS_W + c * CHUNK
        start = jnp.minimum(nominal, LAST_START)
        g_lo = (nominal - start) // 16
        return start, g_lo

    def launch(c):
        slot = c % 2
        start, _ = chunk_start(c)
        dx = pltpu.async_copy(xf.at[pl.ds(start * D_IN, CW)], bufx[slot],
                              semx[slot])
        dl = pltpu.async_copy(lp.at[pl.ds(start, CHUNK)], buflp[slot],
                              seml[slot])
        di = pltpu.async_copy(bid.at[pl.ds(start, CHUNK)], bufid[slot],
                              semi[slot])
        return dx, dl, di

    pending = launch(0)
    for c in range(NCHUNK):
        slot = c % 2
        for d in pending:
            d.wait()
        if c + 1 < NCHUNK:
            pending = launch(c + 1)
        _, g_lo = chunk_start(c)
        bx, bl, bi = bufx[slot], buflp[slot], bufid[slot]

        def row_body(r4, _):
            base = r4 * (4 * D_IN)
            for r in range(4):
                m = [bx[pl.ds(base + r * D_IN + 16 * k, 16)] * wv[k]
                     for k in range(8)]
                a = ((m[0] + m[1]) + (m[2] + m[3])) + \
                    ((m[4] + m[5]) + (m[6] + m[7]))
                ybuf[pl.ds(r4 * 64 + r * 16, 16)] = a
            return 0

        lax.fori_loop(0, CHUNK // 4, row_body, 0)

        def group_body(g, _):
            rb = g * 16
            yb = rb * 16
            g4 = []
            for q in range(4):
                t = [plsc.load_gather(ybuf, [yb + col + 4 * q + j])
                     for j in range(4)]
                g4.append((t[0] + t[1]) + (t[2] + t[3]))
            s = (g4[0] + g4[1]) + (g4[2] + g4[3])
            lpv = bl[pl.ds(rb, 16)]
            s = s * jnp.exp(lpv)
            ids = bi[pl.ds(rb, 16)]
            cs = plsc.cumsum(s)
            ids_next = plsc.load_gather(bi, [rb + jnp.minimum(iota + 1, 15)])
            is_end = (ids != ids_next) | (iota == 15)
            plsc.addupdate_scatter(acc, [ids >> 4, ids & 15], cs, mask=is_end)
            m2 = is_end & (iota < 15)
            plsc.addupdate_scatter(acc, [ids_next >> 4, ids_next & 15], -cs,
                                   mask=m2)
            return 0

        lax.fori_loop(g_lo, GROUPS, group_body, 0)

    pltpu.sync_copy(acc, out.at[wid])


def _merge_kernel(p_ref, b_ref, out_ref):
    out_ref[...] = jnp.sum(p_ref[...], axis=0, keepdims=True) + b_ref[...]


@jax.jit
def kernel(x, node_logprob, batch_ids, y, W, b):
    xf = x.reshape(N_NODES * D_IN)
    bid = batch_ids.astype(jnp.int32)
    w = W.reshape(D_IN)

    mesh = plsc.VectorSubcoreMesh(core_axis_name="c", subcore_axis_name="s")
    sc = functools.partial(
        pl.kernel,
        out_type=jax.ShapeDtypeStruct((NW, 32, 16), jnp.float32),
        mesh=mesh,
        compiler_params=pltpu.CompilerParams(needs_layout_passes=False),
        scratch_types=[
            pltpu.VMEM((CW,), jnp.float32),
            pltpu.VMEM((CW,), jnp.float32),
            pltpu.VMEM((CHUNK,), jnp.float32),
            pltpu.VMEM((CHUNK,), jnp.float32),
            pltpu.VMEM((CHUNK,), jnp.int32),
            pltpu.VMEM((CHUNK,), jnp.int32),
            pltpu.VMEM((D_IN,), jnp.float32),
            pltpu.VMEM((32, 16), jnp.float32),
            pltpu.VMEM((CHUNK * 16,), jnp.float32),
            pltpu.SemaphoreType.DMA,
            pltpu.SemaphoreType.DMA,
            pltpu.SemaphoreType.DMA,
            pltpu.SemaphoreType.DMA,
            pltpu.SemaphoreType.DMA,
            pltpu.SemaphoreType.DMA,
        ],
    )(_sc_kernel)
    partials = sc(xf, node_logprob, bid, w)

    pred = pl.pallas_call(
        _merge_kernel,
        out_shape=jax.ShapeDtypeStruct((1, NUM_GRAPHS), jnp.float32),
    )(partials.reshape(NW, NUM_GRAPHS), b.reshape(1, 1))
    return (pred.reshape(NUM_GRAPHS, 1), y)
